# baseline (speedup 1.0000x reference)
import jax
import jax.numpy as jnp
from jax.experimental import pallas as pl
from jax.experimental.pallas import tpu as pltpu

_R = 32768
_D = 1024
_CROWS = 2048
_NBUF = 6


def _body(pos_ref, x_ref, k_hbm, v_hbm, ox_ref, ok_hbm, ov_hbm,
          bufs, row_buf, in_sems, out_sems, row_sems):
    pos = pos_ref[0]
    nper = _R // _CROWS
    n = 2 * nper

    def src(i):
        arr, j = (k_hbm, i // 2) if i % 2 == 0 else (v_hbm, i // 2)
        return arr.at[pl.ds(j * _CROWS, _CROWS)]

    def dst(i):
        arr, j = (ok_hbm, i // 2) if i % 2 == 0 else (ov_hbm, i // 2)
        return arr.at[pl.ds(j * _CROWS, _CROWS)]

    ready = max(2 * (4095 // _CROWS), 2 * (12292 // _CROWS) + 1)

    ins, outs = [], []
    for j in range(_NBUF):
        c = pltpu.make_async_copy(src(j), bufs.at[j], in_sems.at[j])
        c.start()
        ins.append(c)

    ox_ref[...] = x_ref[...] * x_ref[...]
    row_buf[0, :] = jnp.full((_D,), 100.0, jnp.float32)
    row_buf[1, :] = jnp.full((_D,), 200.0, jnp.float32)

    rows_started = []
    for i in range(n):
        s = i % _NBUF
        ins[i].wait()
        c = pltpu.make_async_copy(bufs.at[s], dst(i), out_sems.at[s])
        c.start()
        outs.append(c)
        ni = i + _NBUF
        if ni < n:
            outs[i].wait()
            c = pltpu.make_async_copy(src(ni), bufs.at[s], in_sems.at[s])
            c.start()
            ins.append(c)
            if i == ready:
                ck = pltpu.make_async_copy(
                    row_buf.at[pl.ds(0, 1)],
                    ok_hbm.at[pl.ds(2048 + pos, 1)], row_sems.at[0])
                cv = pltpu.make_async_copy(
                    row_buf.at[pl.ds(1, 1)],
                    ov_hbm.at[pl.ds(10245 + pos, 1)], row_sems.at[1])
                ck.start()
                cv.start()
                rows_started = [ck, cv]
    for i in range(max(n - _NBUF, 0), n):
        outs[i].wait()
    for c in rows_started:
        c.wait()


def kernel(x, k_cache, v_cache, cache_pos):
    B, S, D = k_cache.shape
    pos = jnp.asarray(cache_pos, jnp.int32).reshape(1)
    kf = k_cache.reshape(B * S, D)
    vf = v_cache.reshape(B * S, D)
    grid_spec = pltpu.PrefetchScalarGridSpec(
        num_scalar_prefetch=1,
        grid=(),
        in_specs=[
            pl.BlockSpec(memory_space=pltpu.VMEM),
            pl.BlockSpec(memory_space=pl.ANY),
            pl.BlockSpec(memory_space=pl.ANY),
        ],
        out_specs=[
            pl.BlockSpec(memory_space=pltpu.VMEM),
            pl.BlockSpec(memory_space=pl.ANY),
            pl.BlockSpec(memory_space=pl.ANY),
        ],
        scratch_shapes=[
            pltpu.VMEM((_NBUF, _CROWS, _D), jnp.float32),
            pltpu.VMEM((2, _D), jnp.float32),
            pltpu.SemaphoreType.DMA((_NBUF,)),
            pltpu.SemaphoreType.DMA((_NBUF,)),
            pltpu.SemaphoreType.DMA((2,)),
        ],
    )
    out_shape = [
        jax.ShapeDtypeStruct(x.shape, x.dtype),
        jax.ShapeDtypeStruct((B * S, D), jnp.float32),
        jax.ShapeDtypeStruct((B * S, D), jnp.float32),
    ]
    ox, ok, ov = pl.pallas_call(
        _body, grid_spec=grid_spec, out_shape=out_shape)(pos, x, kf, vf)
    return (ox, ok.reshape(B, S, D), ov.reshape(B, S, D))


# speedup vs baseline: 1.0023x; 1.0023x over previous
import jax
import jax.numpy as jnp
from jax.experimental import pallas as pl
from jax.experimental.pallas import tpu as pltpu

_R = 32768
_D = 1024
_CROWS = 4096
_NBUF = 3


def _body(pos_ref, x_ref, k_hbm, v_hbm, ox_ref, ok_hbm, ov_hbm,
          bufs, row_buf, in_sems, out_sems, row_sems):
    pos = pos_ref[0]
    nper = _R // _CROWS
    n = 2 * nper

    def src(i):
        arr, j = (k_hbm, i // 2) if i % 2 == 0 else (v_hbm, i // 2)
        return arr.at[pl.ds(j * _CROWS, _CROWS)]

    def dst(i):
        arr, j = (ok_hbm, i // 2) if i % 2 == 0 else (ov_hbm, i // 2)
        return arr.at[pl.ds(j * _CROWS, _CROWS)]

    ready = max(2 * (4095 // _CROWS), 2 * (12292 // _CROWS) + 1)

    ins, outs = [], []
    for j in range(_NBUF):
        c = pltpu.make_async_copy(src(j), bufs.at[j], in_sems.at[j])
        c.start()
        ins.append(c)

    ox_ref[...] = x_ref[...] * x_ref[...]
    row_buf[0, :] = jnp.full((_D,), 100.0, jnp.float32)
    row_buf[1, :] = jnp.full((_D,), 200.0, jnp.float32)

    rows_started = []
    for i in range(n):
        s = i % _NBUF
        ins[i].wait()
        c = pltpu.make_async_copy(bufs.at[s], dst(i), out_sems.at[s])
        c.start()
        outs.append(c)
        ni = i + _NBUF
        if ni < n:
            outs[i].wait()
            c = pltpu.make_async_copy(src(ni), bufs.at[s], in_sems.at[s])
            c.start()
            ins.append(c)
            if i == ready:
                ck = pltpu.make_async_copy(
                    row_buf.at[pl.ds(0, 1)],
                    ok_hbm.at[pl.ds(2048 + pos, 1)], row_sems.at[0])
                cv = pltpu.make_async_copy(
                    row_buf.at[pl.ds(1, 1)],
                    ov_hbm.at[pl.ds(10245 + pos, 1)], row_sems.at[1])
                ck.start()
                cv.start()
                rows_started = [ck, cv]
    for i in range(max(n - _NBUF, 0), n):
        outs[i].wait()
    for c in rows_started:
        c.wait()


def kernel(x, k_cache, v_cache, cache_pos):
    B, S, D = k_cache.shape
    pos = jnp.asarray(cache_pos, jnp.int32).reshape(1)
    kf = k_cache.reshape(B * S, D)
    vf = v_cache.reshape(B * S, D)
    grid_spec = pltpu.PrefetchScalarGridSpec(
        num_scalar_prefetch=1,
        grid=(),
        in_specs=[
            pl.BlockSpec(memory_space=pltpu.VMEM),
            pl.BlockSpec(memory_space=pl.ANY),
            pl.BlockSpec(memory_space=pl.ANY),
        ],
        out_specs=[
            pl.BlockSpec(memory_space=pltpu.VMEM),
            pl.BlockSpec(memory_space=pl.ANY),
            pl.BlockSpec(memory_space=pl.ANY),
        ],
        scratch_shapes=[
            pltpu.VMEM((_NBUF, _CROWS, _D), jnp.float32),
            pltpu.VMEM((2, _D), jnp.float32),
            pltpu.SemaphoreType.DMA((_NBUF,)),
            pltpu.SemaphoreType.DMA((_NBUF,)),
            pltpu.SemaphoreType.DMA((2,)),
        ],
    )
    out_shape = [
        jax.ShapeDtypeStruct(x.shape, x.dtype),
        jax.ShapeDtypeStruct((B * S, D), jnp.float32),
        jax.ShapeDtypeStruct((B * S, D), jnp.float32),
    ]
    ox, ok, ov = pl.pallas_call(
        _body, grid_spec=grid_spec, out_shape=out_shape)(pos, x, kf, vf)
    return (ox, ok.reshape(B, S, D), ov.reshape(B, S, D))


# speedup vs baseline: 1.0041x; 1.0018x over previous
import jax
import jax.numpy as jnp
from jax.experimental import pallas as pl
from jax.experimental.pallas import tpu as pltpu

_R = 32768
_D = 1024
_CROWS = 4096
_NBUF = 3


def _body(pos_ref, x_ref, k_hbm, v_hbm, ox_ref, ok_hbm, ov_hbm,
          bufs, row_buf, in_sems, out_sems, row_sems):
    pos = pos_ref[0]
    nper = _R // _CROWS
    n = 2 * nper

    def src(i):
        arr, j = (k_hbm, i // 2) if i % 2 == 0 else (v_hbm, i // 2)
        return arr.at[pl.ds(j * _CROWS, _CROWS)]

    def dst(i):
        arr, j = (ok_hbm, i // 2) if i % 2 == 0 else (ov_hbm, i // 2)
        return arr.at[pl.ds(j * _CROWS, _CROWS)]

    ready = max(2 * (4095 // _CROWS), 2 * (12287 // _CROWS) + 1)
    k_in = pos < 2048
    v_in = pos + 5 < 2048

    ins, outs = [], []
    for j in range(_NBUF):
        c = pltpu.make_async_copy(src(j), bufs.at[j], in_sems.at[j])
        c.start()
        ins.append(c)

    ox_ref[...] = x_ref[...] * x_ref[...]
    row_buf[0, :] = jnp.full((_D,), 100.0, jnp.float32)
    row_buf[1, :] = jnp.full((_D,), 200.0, jnp.float32)

    rows_started = []
    for i in range(n):
        s = i % _NBUF
        ins[i].wait()
        c = pltpu.make_async_copy(bufs.at[s], dst(i), out_sems.at[s])
        c.start()
        outs.append(c)
        ni = i + _NBUF
        if ni < n:
            outs[i].wait()
            c = pltpu.make_async_copy(src(ni), bufs.at[s], in_sems.at[s])
            c.start()
            ins.append(c)
            if i == ready:
                ck = pltpu.make_async_copy(
                    row_buf.at[pl.ds(0, 1)],
                    ok_hbm.at[pl.ds(2048 + pos, 1)], row_sems.at[0])
                cv = pltpu.make_async_copy(
                    row_buf.at[pl.ds(1, 1)],
                    ov_hbm.at[pl.ds(10245 + pos, 1)], row_sems.at[1])

                @pl.when(k_in)
                def _():
                    ck.start()

                @pl.when(v_in)
                def _():
                    cv.start()

                rows_started = [(k_in, ck), (v_in, cv)]
    for i in range(max(n - _NBUF, 0), n):
        outs[i].wait()
    for cond, c in rows_started:
        @pl.when(cond)
        def _():
            c.wait()


def kernel(x, k_cache, v_cache, cache_pos):
    B, S, D = k_cache.shape
    pos = jnp.asarray(cache_pos, jnp.int32).reshape(1)
    kf = k_cache.reshape(B * S, D)
    vf = v_cache.reshape(B * S, D)
    grid_spec = pltpu.PrefetchScalarGridSpec(
        num_scalar_prefetch=1,
        grid=(),
        in_specs=[
            pl.BlockSpec(memory_space=pltpu.VMEM),
            pl.BlockSpec(memory_space=pl.ANY),
            pl.BlockSpec(memory_space=pl.ANY),
        ],
        out_specs=[
            pl.BlockSpec(memory_space=pltpu.VMEM),
            pl.BlockSpec(memory_space=pl.ANY),
            pl.BlockSpec(memory_space=pl.ANY),
        ],
        scratch_shapes=[
            pltpu.VMEM((_NBUF, _CROWS, _D), jnp.float32),
            pltpu.VMEM((2, _D), jnp.float32),
            pltpu.SemaphoreType.DMA((_NBUF,)),
            pltpu.SemaphoreType.DMA((_NBUF,)),
            pltpu.SemaphoreType.DMA((2,)),
        ],
    )
    out_shape = [
        jax.ShapeDtypeStruct(x.shape, x.dtype),
        jax.ShapeDtypeStruct((B * S, D), jnp.float32),
        jax.ShapeDtypeStruct((B * S, D), jnp.float32),
    ]
    ox, ok, ov = pl.pallas_call(
        _body, grid_spec=grid_spec, out_shape=out_shape)(pos, x, kf, vf)
    return (ox, ok.reshape(B, S, D), ov.reshape(B, S, D))
